# baseline (speedup 1.0000x reference)
import functools

import jax
import jax.numpy as jnp
from jax import lax
from jax.experimental import pallas as pl
from jax.experimental.pallas import tpu as pltpu


def _ln_stream_kernel(x_hbm, w_ref, b_ref, o_hbm, in_buf, out_buf, in_sem,
                      out_sem, *, eps, inv_c, chunk, nchunks):
    i = pl.program_id(0)
    base = i * (chunk * nchunks)

    def in_copy(k):
        return pltpu.make_async_copy(
            x_hbm.at[pl.ds(base + k * chunk, chunk), :],
            in_buf.at[k % 2],
            in_sem.at[k % 2],
        )

    def out_copy(k):
        return pltpu.make_async_copy(
            out_buf.at[k % 2],
            o_hbm.at[pl.ds(base + k * chunk, chunk), :],
            out_sem.at[k % 2],
        )

    w = w_ref[...]
    b = b_ref[...]
    in_copy(0).start()
    for k in range(nchunks):
        if k + 1 < nchunks:
            in_copy(k + 1).start()
        in_copy(k).wait()
        if k >= 2:
            out_copy(k - 2).wait()
        x = in_buf[k % 2]
        s = jnp.sum(x, axis=-1, keepdims=True)
        sq = jnp.sum(x * x, axis=-1, keepdims=True)
        mu = s * inv_c
        var = sq * inv_c - mu * mu
        inv = lax.rsqrt(var + eps)
        out_buf[k % 2] = (x - mu) * inv * w + b
        out_copy(k).start()
    if nchunks >= 2:
        out_copy(nchunks - 2).wait()
    out_copy(nchunks - 1).wait()


def kernel(x, weight, bias, *, eps=1e-6):
    c = x.shape[-1]
    lead = x.shape[:-1]
    x2d = x.reshape(-1, c)
    rows = x2d.shape[0]

    ncores = 2
    chunk = 1024
    nchunks = rows // (ncores * chunk)

    kernel_fn = functools.partial(
        _ln_stream_kernel, eps=eps, inv_c=1.0 / c, chunk=chunk, nchunks=nchunks)
    y2d = pl.pallas_call(
        kernel_fn,
        out_shape=jax.ShapeDtypeStruct((rows, c), x.dtype),
        grid=(ncores,),
        in_specs=[
            pl.BlockSpec(memory_space=pl.ANY),
            pl.BlockSpec((1, c), lambda i: (0, 0)),
            pl.BlockSpec((1, c), lambda i: (0, 0)),
        ],
        out_specs=pl.BlockSpec(memory_space=pl.ANY),
        scratch_shapes=[
            pltpu.VMEM((2, chunk, c), x.dtype),
            pltpu.VMEM((2, chunk, c), x.dtype),
            pltpu.SemaphoreType.DMA((2,)),
            pltpu.SemaphoreType.DMA((2,)),
        ],
        compiler_params=pltpu.CompilerParams(
            dimension_semantics=("parallel",),
            vmem_limit_bytes=48 * 1024 * 1024,
        ),
    )(x2d, weight.reshape(1, c), bias.reshape(1, c))
    return y2d.reshape(*lead, c)


# speedup vs baseline: 1.1378x; 1.1378x over previous
import functools

import jax
import jax.numpy as jnp
from jax import lax
from jax.experimental import pallas as pl
from jax.experimental.pallas import tpu as pltpu


def _ln_stream_kernel(x_hbm, w_ref, b_ref, o_hbm, buf, in_sem, out_sem, *,
                      eps, inv_c, chunk, nchunks):
    i = pl.program_id(0)
    base = i * (chunk * nchunks)

    def in_copy(k):
        return pltpu.make_async_copy(
            x_hbm.at[pl.ds(base + k * chunk, chunk), :],
            buf.at[k],
            in_sem.at[k],
        )

    def out_copy(k):
        return pltpu.make_async_copy(
            buf.at[k],
            o_hbm.at[pl.ds(base + k * chunk, chunk), :],
            out_sem.at[k],
        )

    for k in range(nchunks):
        in_copy(k).start()
    w = w_ref[...]
    b = b_ref[...]
    for k in range(nchunks):
        in_copy(k).wait()
        x = buf[k]
        s = jnp.sum(x, axis=-1, keepdims=True)
        sq = jnp.sum(x * x, axis=-1, keepdims=True)
        mu = s * inv_c
        var = sq * inv_c - mu * mu
        inv = lax.rsqrt(var + eps)
        buf[k] = (x - mu) * inv * w + b
        out_copy(k).start()
    for k in range(nchunks):
        out_copy(k).wait()


def kernel(x, weight, bias, *, eps=1e-6):
    c = x.shape[-1]
    lead = x.shape[:-1]
    x2d = x.reshape(-1, c)
    rows = x2d.shape[0]

    ncores = 2
    nchunks = 4
    chunk = rows // (ncores * nchunks)

    kernel_fn = functools.partial(
        _ln_stream_kernel, eps=eps, inv_c=1.0 / c, chunk=chunk, nchunks=nchunks)
    y2d = pl.pallas_call(
        kernel_fn,
        out_shape=jax.ShapeDtypeStruct((rows, c), x.dtype),
        grid=(ncores,),
        in_specs=[
            pl.BlockSpec(memory_space=pl.ANY),
            pl.BlockSpec((1, c), lambda i: (0, 0)),
            pl.BlockSpec((1, c), lambda i: (0, 0)),
        ],
        out_specs=pl.BlockSpec(memory_space=pl.ANY),
        scratch_shapes=[
            pltpu.VMEM((nchunks, chunk, c), x.dtype),
            pltpu.SemaphoreType.DMA((nchunks,)),
            pltpu.SemaphoreType.DMA((nchunks,)),
        ],
        compiler_params=pltpu.CompilerParams(
            dimension_semantics=("parallel",),
            vmem_limit_bytes=48 * 1024 * 1024,
        ),
    )(x2d, weight.reshape(1, c), bias.reshape(1, c))
    return y2d.reshape(*lead, c)
